# baseline (speedup 1.0000x reference)
import functools

import jax
import jax.numpy as jnp
from jax import lax
from jax.experimental import pallas as pl
from jax.experimental.pallas import tpu as pltpu
from jax.experimental.pallas import tpu_sc as plsc

B = 16384
D = 32
NC = 2
NS = 16
NW = NC * NS
BPW = B // NW
NSLOT = 4
NRUNMAX = BPW + 16
LAST_Q = 7811
TAIL_BASE = 999872
LANES = 16


def _splat(x):
    return jnp.full((LANES,), x, jnp.int32)


def _get1(ref, i):
    return plsc.load_gather(ref, [_splat(i)])[0]


def _pass_body(tT, s, p, tail, emb, sv, pv, runq, runs, scr, tailscr,
               vst_idx, vst_val, sems, semt, sem_sc):
    wid = lax.axis_index("s") * NC + lax.axis_index("c")
    base = wid * BPW
    pltpu.sync_copy(s.at[pl.ds(base, BPW)], sv)
    pltpu.sync_copy(p.at[pl.ds(base, BPW)], pv)
    pltpu.sync_copy(tail, tailscr)
    lane = lax.iota(jnp.int32, LANES)

    nr = jnp.int32(0)
    for g in range(BPW // LANES):
        rv = sv[pl.ds(g * LANES, LANES)]
        qv = lax.shift_right_logical(rv, 7)
        prev_pos = g * LANES + lane - 1
        prev_pos = jnp.maximum(prev_pos, 0)
        prevq = lax.shift_right_logical(plsc.load_gather(sv, [prev_pos]), 7)
        neq = qv != prevq
        if g == 0:
            neq = neq | (lane == 0)
        plsc.store_compressed(runq.at[pl.ds(nr, LANES)], qv, mask=neq)
        plsc.store_compressed(runs.at[pl.ds(nr, LANES)],
                              g * LANES + lane, mask=neq)
        nr = nr + plsc.all_reduce_population_count(neq)[0]
    runs[pl.ds(nr, LANES)] = _splat(BPW)

    def issue(slot, q):
        qq = jnp.minimum(q, LAST_Q)
        off = pl.multiple_of(lax.shift_left(qq, 7), 128)
        for tr in range(4):
            pltpu.async_copy(
                tT.at[pl.ds(tr * 8, 8), pl.ds(off, 128)],
                scr.at[slot, tr], sems.at[slot])

    for k in range(NSLOT):
        @pl.when(k < nr)
        def _():
            issue(k, _get1(runq, k))

    def run_body(j, carry):
        slot = j & (NSLOT - 1)
        for tr in range(4):
            pltpu.make_async_copy(
                tT.at[pl.ds(0, 8), pl.ds(0, 128)],
                scr.at[slot, tr], sems.at[slot]).wait()

        lo = _get1(runs, j)
        hi = _get1(runs, j + 1)

        def ent_body(e, ecarry):
            r = _get1(sv, e)
            b = _get1(pv, e)
            q = lax.shift_right_logical(r, 7)
            col = _splat(r & 127)
            tcol = _splat(jnp.maximum(r - TAIL_BASE, 0))
            slat = _splat(slot)
            in_tail = r >= TAIL_BASE
            lo16 = plsc.load_gather(scr, [slat, lane >> 3, lane & 7, col])
            hi16 = plsc.load_gather(
                scr, [slat, 2 + (lane >> 3), lane & 7, col])
            tl16 = plsc.load_gather(tailscr, [lane, tcol])
            th16 = plsc.load_gather(tailscr, [lane + 16, tcol])
            vlo = jnp.where(in_tail, tl16, lo16)
            vhi = jnp.where(in_tail, th16, hi16)
            row = lax.shift_right_logical(e, 2)
            cb = (e & 3) * D
            vst_idx[row, pl.ds(cb, LANES)] = lane * B + b
            vst_idx[row, pl.ds(cb + LANES, LANES)] = (lane + LANES) * B + b
            vst_val[row, pl.ds(cb, LANES)] = vlo
            vst_val[row, pl.ds(cb + LANES, LANES)] = vhi
            return ecarry

        lax.fori_loop(lo, hi, ent_body, 0)

        @pl.when(j + NSLOT < nr)
        def _():
            issue(slot, _get1(runq, j + NSLOT))
        return carry

    lax.fori_loop(0, nr, run_body, 0)

    cps = []
    for row in range(BPW * D // 128):
        cps.append(pltpu.async_copy(
            vst_val.at[row], emb.at[vst_idx.at[row]], sem_sc))
    for cp in cps:
        cp.wait()


def _mul_body(eu, ei, out, du, di):
    wid = lax.axis_index("s") * NC + lax.axis_index("c")
    base = wid * (B * D // NW)
    pltpu.sync_copy(eu.at[pl.ds(base, B * D // NW)], du)
    pltpu.sync_copy(ei.at[pl.ds(base, B * D // NW)], di)
    for g in range(B * D // NW // LANES):
        sl = pl.ds(g * LANES, LANES)
        du[sl] = du[sl] * di[sl]
    pltpu.sync_copy(du, out.at[pl.ds(base, B * D // NW)])


def _make_pass():
    mesh = plsc.VectorSubcoreMesh(core_axis_name="c", subcore_axis_name="s")
    return functools.partial(
        pl.kernel,
        mesh=mesh,
        out_type=jax.ShapeDtypeStruct((B * D,), jnp.float32),
        compiler_params=pltpu.CompilerParams(needs_layout_passes=False),
        scratch_types=[
            pltpu.VMEM((BPW,), jnp.int32),
            pltpu.VMEM((BPW,), jnp.int32),
            pltpu.VMEM((NRUNMAX,), jnp.int32),
            pltpu.VMEM((NRUNMAX + 1,), jnp.int32),
            pltpu.VMEM((NSLOT, 4, 8, 128), jnp.float32),
            pltpu.VMEM((D, 128), jnp.float32),
            pltpu.VMEM((BPW * D // 128, 128), jnp.int32),
            pltpu.VMEM((BPW * D // 128, 128), jnp.float32),
            pltpu.SemaphoreType.DMA((NSLOT,)),
            pltpu.SemaphoreType.DMA,
            pltpu.SemaphoreType.DMA,
        ],
    )(_pass_body)


def _make_mul():
    mesh = plsc.VectorSubcoreMesh(core_axis_name="c", subcore_axis_name="s")
    return functools.partial(
        pl.kernel,
        mesh=mesh,
        out_type=jax.ShapeDtypeStruct((B * D,), jnp.float32),
        scratch_types=[
            pltpu.VMEM((B * D // NW,), jnp.float32),
            pltpu.VMEM((B * D // NW,), jnp.float32),
        ],
    )(_mul_body)


def kernel(user_table, item_table, user_indices, item_indices):
    iota = jnp.arange(B, dtype=jnp.int32)
    su, pu = lax.sort((user_indices, iota), num_keys=1)
    si, pi = lax.sort((item_indices, iota), num_keys=1)
    utT = user_table.T
    itT = item_table.T
    tail_u = user_table[TAIL_BASE:, :].T
    tail_i = item_table[TAIL_BASE:, :].T
    pk = _make_pass()
    emb_u = pk(utT, su, pu, tail_u)
    emb_i = pk(itT, si, pi, tail_i)
    out = _make_mul()(emb_u, emb_i)
    return out.reshape(D, B).T
